# Initial kernel scaffold; baseline (speedup 1.0000x reference)
import jax
import jax.numpy as jnp
from jax.experimental import pallas as pl


def kernel(item_sequences, item_ids, emb_weight, bias_weight):
    raise NotImplementedError("write your pallas kernel here")


# speedup vs baseline: 1.6260x; 1.6260x over previous
import functools

import jax
import jax.numpy as jnp
from jax import lax
from jax.experimental import pallas as pl
from jax.experimental.pallas import tpu as pltpu
from jax.experimental.pallas import tpu_sc as plsc

B = 200
L = 200
D = 64
LANES = 16


def _sc_dot_bias(seqs, ids, emb, bias_flat):
    info = plsc.get_sparse_core_info()
    nw = info.num_cores * info.num_subcores
    nb = -(-B // nw)

    mesh = plsc.VectorSubcoreMesh(core_axis_name="c", subcore_axis_name="s")

    @functools.partial(
        pl.kernel,
        mesh=mesh,
        compiler_params=pltpu.CompilerParams(
            needs_layout_passes=False, use_tc_tiling_on_sc=False),
        out_type=[
            jax.ShapeDtypeStruct((B, L), jnp.float32),
            jax.ShapeDtypeStruct((B, L), jnp.float32),
        ],
        scratch_types=[
            pltpu.VMEM((L,), jnp.int32),
            pltpu.VMEM((L,), jnp.int32),
            pltpu.VMEM((L, D), jnp.float32),
            pltpu.VMEM((L, D), jnp.float32),
            pltpu.VMEM((L,), jnp.float32),
            pltpu.VMEM((208 * LANES,), jnp.float32),
            pltpu.VMEM((208,), jnp.float32),
            pltpu.SemaphoreType.DMA,
        ],
    )
    def k(seq_hbm, ids_hbm, emb_hbm, bias_hbm, dot_hbm, biasg_hbm,
          seqi_v, tgti_v, seen_v, tgt_v, bias_v, pp_v, dot_v, sem):
        wid = lax.axis_index("s") * info.num_cores + lax.axis_index("c")

        def one_batch(b):
            pltpu.sync_copy(seq_hbm.at[b], seqi_v)
            pltpu.sync_copy(ids_hbm.at[b], tgti_v)
            cps = [
                pltpu.async_copy(emb_hbm.at[seqi_v.at[pl.ds(0, 128)]],
                                 seen_v.at[pl.ds(0, 128), :], sem),
                pltpu.async_copy(emb_hbm.at[seqi_v.at[pl.ds(128, 72)]],
                                 seen_v.at[pl.ds(128, 72), :], sem),
                pltpu.async_copy(emb_hbm.at[tgti_v.at[pl.ds(0, 128)]],
                                 tgt_v.at[pl.ds(0, 128), :], sem),
                pltpu.async_copy(emb_hbm.at[tgti_v.at[pl.ds(128, 72)]],
                                 tgt_v.at[pl.ds(128, 72), :], sem),
                pltpu.async_copy(bias_hbm.at[tgti_v.at[pl.ds(0, 128)]],
                                 bias_v.at[pl.ds(0, 128)], sem),
                pltpu.async_copy(bias_hbm.at[tgti_v.at[pl.ds(128, 72)]],
                                 bias_v.at[pl.ds(128, 72)], sem),
            ]
            for cp in cps:
                cp.wait()

            def body(l, acc):
                a0, a1, a2, a3 = acc
                a0 = a0 + seen_v[l, pl.ds(0, 16)]
                a1 = a1 + seen_v[l, pl.ds(16, 16)]
                a2 = a2 + seen_v[l, pl.ds(32, 16)]
                a3 = a3 + seen_v[l, pl.ds(48, 16)]
                p = (a0 * tgt_v[l, pl.ds(0, 16)]
                     + a1 * tgt_v[l, pl.ds(16, 16)]
                     + a2 * tgt_v[l, pl.ds(32, 16)]
                     + a3 * tgt_v[l, pl.ds(48, 16)])
                pp_v[pl.ds(l * LANES, LANES)] = p
                return (a0, a1, a2, a3)

            z = jnp.zeros((LANES,), jnp.float32)
            lax.fori_loop(jnp.int32(0), jnp.int32(L), body, (z, z, z, z))

            def body2(g, carry):
                l0 = g * LANES
                rows = (l0 + lax.iota(jnp.int32, LANES)) * LANES
                tot = jnp.zeros((LANES,), jnp.float32)
                for d16 in range(LANES):
                    tot = tot + plsc.load_gather(pp_v, [rows + d16])
                dot_v[pl.ds(l0, LANES)] = tot
                return carry

            lax.fori_loop(jnp.int32(0), jnp.int32(13), body2, 0)
            pltpu.sync_copy(dot_v.at[pl.ds(0, L)], dot_hbm.at[b])
            pltpu.sync_copy(bias_v, biasg_hbm.at[b])

        def outer(i, carry):
            b = wid + nw * i

            @pl.when(b < B)
            def _():
                one_batch(b)

            return carry

        lax.fori_loop(jnp.int32(0), jnp.int32(nb), outer, 0)

    return k(seqs, ids, emb, bias_flat)


def _tc_broadcast(dot, biasg):
    gi = 8

    def body(dot_ref, bias_ref, out_ref):
        out_ref[...] = dot_ref[...][None, :, :] + bias_ref[...][:, :, None]

    return pl.pallas_call(
        body,
        grid=(B // gi,),
        in_specs=[
            pl.BlockSpec((B, L), lambda i: (jnp.int32(0), jnp.int32(0))),
            pl.BlockSpec((gi, L), lambda i: (i, jnp.int32(0))),
        ],
        out_specs=pl.BlockSpec(
            (gi, B, L), lambda i: (i, jnp.int32(0), jnp.int32(0))),
        out_shape=jax.ShapeDtypeStruct((B, B, L), jnp.float32),
    )(dot, biasg)


def kernel(item_sequences, item_ids, emb_weight, bias_weight):
    seqs = item_sequences.astype(jnp.int32)
    ids = item_ids.astype(jnp.int32)
    bias_flat = bias_weight.reshape((-1,))
    dot, biasg = _sc_dot_bias(seqs, ids, emb_weight, bias_flat)
    return _tc_broadcast(dot, biasg)


# speedup vs baseline: 1.8054x; 1.1103x over previous
import functools

import jax
import jax.numpy as jnp
from jax import lax
from jax.experimental import pallas as pl
from jax.experimental.pallas import tpu as pltpu
from jax.experimental.pallas import tpu_sc as plsc

B = 200
L = 200
D = 64
LANES = 16
_i32 = jnp.int32


def _sc_dot_bias(seqs, ids, emb, bias_flat):
    info = plsc.get_sparse_core_info()
    nw = info.num_cores * info.num_subcores
    nb = -(-B // nw)

    mesh = plsc.VectorSubcoreMesh(core_axis_name="c", subcore_axis_name="s")

    @functools.partial(
        pl.kernel,
        mesh=mesh,
        compiler_params=pltpu.CompilerParams(
            needs_layout_passes=False, use_tc_tiling_on_sc=False),
        out_type=[
            jax.ShapeDtypeStruct((B, L), jnp.float32),
            jax.ShapeDtypeStruct((B, L), jnp.float32),
        ],
        scratch_types=[
            pltpu.VMEM((nb, 2, L), jnp.int32),
            pltpu.VMEM((2, L, D), jnp.float32),
            pltpu.VMEM((2, L, D), jnp.float32),
            pltpu.VMEM((2, L), jnp.float32),
            pltpu.VMEM((208 * LANES,), jnp.float32),
            pltpu.VMEM((208,), jnp.float32),
            pltpu.SemaphoreType.DMA,
            pltpu.SemaphoreType.DMA,
            pltpu.SemaphoreType.DMA,
        ],
    )
    def k(seq_hbm, ids_hbm, emb_hbm, bias_hbm, dot_hbm, biasg_hbm,
          idx_v, seen_v, tgt_v, bias_v, pp_v, dot_v, isem, gsem0, gsem1):
        wid = lax.axis_index("s") * info.num_cores + lax.axis_index("c")
        gsems = [gsem0, gsem1]

        bs = [wid + nw * i for i in range(nb)]

        for i in range(nb):
            @pl.when(bs[i] < B)
            def _(i=i):
                pltpu.async_copy(seq_hbm.at[bs[i]], idx_v.at[_i32(i), _i32(0)], isem)
                pltpu.async_copy(ids_hbm.at[bs[i]], idx_v.at[_i32(i), _i32(1)], isem)

        for i in range(nb):
            @pl.when(bs[i] < B)
            def _(i=i):
                pltpu.make_async_copy(seq_hbm.at[bs[i]], idx_v.at[_i32(i), _i32(0)], isem).wait()
                pltpu.make_async_copy(ids_hbm.at[bs[i]], idx_v.at[_i32(i), _i32(1)], isem).wait()

        def gather_list(i, s):
            sem = gsems[s]
            return [
                (emb_hbm.at[idx_v.at[_i32(i), _i32(0), pl.ds(0, 128)]],
                 seen_v.at[_i32(s), pl.ds(0, 128), :], sem),
                (emb_hbm.at[idx_v.at[_i32(i), _i32(0), pl.ds(128, 72)]],
                 seen_v.at[_i32(s), pl.ds(128, 72), :], sem),
                (emb_hbm.at[idx_v.at[_i32(i), _i32(1), pl.ds(0, 128)]],
                 tgt_v.at[_i32(s), pl.ds(0, 128), :], sem),
                (emb_hbm.at[idx_v.at[_i32(i), _i32(1), pl.ds(128, 72)]],
                 tgt_v.at[_i32(s), pl.ds(128, 72), :], sem),
                (bias_hbm.at[idx_v.at[_i32(i), _i32(1), pl.ds(0, 128)]],
                 bias_v.at[_i32(s), pl.ds(0, 128)], sem),
                (bias_hbm.at[idx_v.at[_i32(i), _i32(1), pl.ds(128, 72)]],
                 bias_v.at[_i32(s), pl.ds(128, 72)], sem),
            ]

        def fire(i, s):
            @pl.when(bs[i] < B)
            def _():
                for src, dst, sem in gather_list(i, s):
                    pltpu.async_copy(src, dst, sem)

        def drain(i, s):
            @pl.when(bs[i] < B)
            def _():
                for src, dst, sem in gather_list(i, s):
                    pltpu.make_async_copy(src, dst, sem).wait()

        def compute(i, s):
            b = bs[i]

            @pl.when(b < B)
            def _():
                def body(l, acc):
                    a0, a1, a2, a3 = acc
                    a0 = a0 + seen_v[_i32(s), l, pl.ds(0, 16)]
                    a1 = a1 + seen_v[_i32(s), l, pl.ds(16, 16)]
                    a2 = a2 + seen_v[_i32(s), l, pl.ds(32, 16)]
                    a3 = a3 + seen_v[_i32(s), l, pl.ds(48, 16)]
                    p = (a0 * tgt_v[_i32(s), l, pl.ds(0, 16)]
                         + a1 * tgt_v[_i32(s), l, pl.ds(16, 16)]
                         + a2 * tgt_v[_i32(s), l, pl.ds(32, 16)]
                         + a3 * tgt_v[_i32(s), l, pl.ds(48, 16)])
                    pp_v[pl.ds(l * LANES, LANES)] = p
                    return (a0, a1, a2, a3)

                z = jnp.zeros((LANES,), jnp.float32)
                lax.fori_loop(jnp.int32(0), jnp.int32(L), body, (z, z, z, z))

                def body2(g, carry):
                    l0 = g * LANES
                    rows = (l0 + lax.iota(jnp.int32, LANES)) * LANES
                    tot = jnp.zeros((LANES,), jnp.float32)
                    for d16 in range(LANES):
                        tot = tot + plsc.load_gather(pp_v, [rows + d16])
                    dot_v[pl.ds(l0, LANES)] = tot
                    return carry

                lax.fori_loop(jnp.int32(0), jnp.int32(13), body2, 0)
                pltpu.sync_copy(dot_v.at[pl.ds(0, L)], dot_hbm.at[b])
                pltpu.sync_copy(bias_v.at[_i32(s)], biasg_hbm.at[b])

        fire(0, 0)
        for i in range(nb):
            s = i % 2
            drain(i, s)
            if i + 1 < nb:
                fire(i + 1, (i + 1) % 2)
            compute(i, s)

    return k(seqs, ids, emb, bias_flat)


def _tc_broadcast(dot, biasg):
    gi = 8

    def body(dot_ref, bias_ref, out_ref):
        out_ref[...] = dot_ref[...][None, :, :] + bias_ref[...][:, :, None]

    return pl.pallas_call(
        body,
        grid=(B // gi,),
        in_specs=[
            pl.BlockSpec((B, L), lambda i: (jnp.int32(0), jnp.int32(0))),
            pl.BlockSpec((gi, L), lambda i: (i, jnp.int32(0))),
        ],
        out_specs=pl.BlockSpec(
            (gi, B, L), lambda i: (i, jnp.int32(0), jnp.int32(0))),
        out_shape=jax.ShapeDtypeStruct((B, B, L), jnp.float32),
    )(dot, biasg)


def kernel(item_sequences, item_ids, emb_weight, bias_weight):
    seqs = item_sequences.astype(jnp.int32)
    ids = item_ids.astype(jnp.int32)
    bias_flat = bias_weight.reshape((-1,))
    dot, biasg = _sc_dot_bias(seqs, ids, emb_weight, bias_flat)
    return _tc_broadcast(dot, biasg)


# speedup vs baseline: 2.0707x; 1.1470x over previous
import functools

import jax
import jax.numpy as jnp
from jax import lax
from jax.experimental import pallas as pl
from jax.experimental.pallas import tpu as pltpu
from jax.experimental.pallas import tpu_sc as plsc

B = 200
L = 200
D = 64
V = 100000
LANES = 16
LP = 256
_i32 = jnp.int32


def _sc_partials(seqs, ids, embT, bias_flat):
    info = plsc.get_sparse_core_info()
    nc, ns = info.num_cores, info.num_subcores
    nw = nc * ns

    mesh = plsc.VectorSubcoreMesh(core_axis_name="c", subcore_axis_name="s")

    @functools.partial(
        pl.kernel,
        mesh=mesh,
        compiler_params=pltpu.CompilerParams(
            needs_layout_passes=False, use_tc_tiling_on_sc=True),
        out_type=[
            jax.ShapeDtypeStruct((D, B, LP), jnp.float32),
            jax.ShapeDtypeStruct((B, LP), jnp.float32),
        ],
        scratch_types=[
            pltpu.VMEM((V,), jnp.float32),
            pltpu.VMEM((2, 16, L), jnp.int32),
            pltpu.VMEM((2, 16, L), jnp.int32),
            pltpu.VMEM((2, 16, LP), jnp.float32),
            pltpu.VMEM((L,), jnp.int32),
            pltpu.VMEM((LP,), jnp.float32),
            pltpu.SemaphoreType.DMA,
            pltpu.SemaphoreType.DMA,
            pltpu.SemaphoreType.DMA,
        ],
    )
    def k(seq_hbm, ids_hbm, embT_hbm, bias_hbm, parts_hbm, biasg_hbm,
          col_v, sidx_v, tidx_v, part_v, bidx_v, bval_v, sem, isem, psem):
        c = lax.axis_index("c")
        s = lax.axis_index("s")
        wid = s * nc + c

        def bias_batch(i, carry):
            b = wid + nw * i

            @pl.when(b < B)
            def _():
                pltpu.sync_copy(ids_hbm.at[b], bidx_v)
                pltpu.async_copy(bias_hbm.at[bidx_v.at[pl.ds(0, 128)]],
                                 bval_v.at[pl.ds(0, 128)], sem).wait()
                pltpu.async_copy(bias_hbm.at[bidx_v.at[pl.ds(128, 72)]],
                                 bval_v.at[pl.ds(128, 72)], sem).wait()
                pltpu.sync_copy(bval_v, biasg_hbm.at[b])

            return carry

        lax.fori_loop(_i32(0), _i32(7), bias_batch, 0)

        NI = 8

        def chunk_b0(g):
            return pl.multiple_of(jnp.minimum(g * _i32(16), _i32(B - 16)), 8)

        def idx_copies(g, slot):
            b0 = chunk_b0(g)
            return [
                (seq_hbm.at[pl.ds(b0, 16)], sidx_v.at[slot]),
                (ids_hbm.at[pl.ds(b0, 16)], tidx_v.at[slot]),
            ]

        def quad_body_for(slot):
            def quad_body(rq, carry2):
                rows = [rq * _i32(NI) + _i32(j) for j in range(NI)]
                cc = [jnp.float32(0.0)] * NI
                cc_tail = [jnp.float32(0.0)] * NI
                for grp in range(13):
                    l0 = grp * 16 if grp < 12 else L - 16
                    xs = [plsc.load_gather(
                        col_v, [sidx_v[slot, r, pl.ds(l0, 16)]])
                        for r in rows]
                    ts = [plsc.load_gather(
                        col_v, [tidx_v[slot, r, pl.ds(l0, 16)]])
                        for r in rows]
                    pres = [plsc.cumsum(x) for x in xs]
                    cums = []
                    for j in range(NI):
                        if grp < 12:
                            cums.append(pres[j] + cc[j])
                            if grp == 11:
                                cc_tail[j] = cc[j] + pres[j][7]
                            cc[j] = cc[j] + pres[j][15]
                        else:
                            cums.append(pres[j] + cc_tail[j])
                    for j in range(NI):
                        part_v[slot, rows[j], pl.ds(l0, 16)] = cums[j] * ts[j]
                return carry2

            return quad_body

        for dpass in range(2):
            d = c * _i32(32) + _i32(16 * dpass) + s
            pltpu.sync_copy(embT_hbm.at[d], col_v)
            for src_, dst_ in idx_copies(_i32(0), _i32(0)):
                pltpu.async_copy(src_, dst_, isem)

            def chunk_body(g, carry):
                slot = lax.rem(g, _i32(2))
                for src_, dst_ in idx_copies(g, slot):
                    pltpu.make_async_copy(src_, dst_, isem).wait()

                @pl.when(g < _i32(12))
                def _():
                    for src_, dst_ in idx_copies(g + 1, _i32(1) - slot):
                        pltpu.async_copy(src_, dst_, isem)

                @pl.when(g >= _i32(2))
                def _():
                    pltpu.make_async_copy(
                        part_v.at[slot],
                        parts_hbm.at[d, pl.ds(chunk_b0(g), 16)], psem).wait()

                lax.fori_loop(_i32(0), _i32(2), quad_body_for(slot), 0)
                pltpu.async_copy(
                    part_v.at[slot],
                    parts_hbm.at[d, pl.ds(chunk_b0(g), 16)], psem)
                return carry

            lax.fori_loop(_i32(0), _i32(13), chunk_body, 0)

            for gg in (11, 12):
                slot = _i32(gg % 2)
                pltpu.make_async_copy(
                    part_v.at[slot],
                    parts_hbm.at[d, pl.ds(chunk_b0(_i32(gg)), 16)],
                    psem).wait()

    return k(seqs, ids, embT, bias_flat)


def _tc_broadcast(parts, biasg):
    gi = 8

    def body(parts_ref, bias_ref, out_ref, dot_v):
        @pl.when(pl.program_id(0) == 0)
        def _():
            acc = parts_ref[0]
            for d in range(1, D):
                acc = acc + parts_ref[d]
            dot_v[...] = acc

        dot = dot_v[:, :L]
        out_ref[...] = dot[None, :, :] + bias_ref[:, :L][:, :, None]

    return pl.pallas_call(
        body,
        grid=(B // gi,),
        in_specs=[
            pl.BlockSpec((D, B, LP), lambda i: (_i32(0), _i32(0), _i32(0))),
            pl.BlockSpec((gi, LP), lambda i: (i, _i32(0))),
        ],
        out_specs=pl.BlockSpec(
            (gi, B, L), lambda i: (i, _i32(0), _i32(0))),
        out_shape=jax.ShapeDtypeStruct((B, B, L), jnp.float32),
        scratch_shapes=[pltpu.VMEM((B, LP), jnp.float32)],
    )(parts, biasg)


def kernel(item_sequences, item_ids, emb_weight, bias_weight):
    seqs = item_sequences.astype(jnp.int32)
    ids = item_ids.astype(jnp.int32)
    bias_flat = bias_weight.reshape((-1,))
    parts, biasg = _sc_partials(seqs, ids, emb_weight.T, bias_flat)
    return _tc_broadcast(parts, biasg)


# speedup vs baseline: 2.2561x; 1.0895x over previous
import functools

import jax
import jax.numpy as jnp
from jax import lax
from jax.experimental import pallas as pl
from jax.experimental.pallas import tpu as pltpu
from jax.experimental.pallas import tpu_sc as plsc

B = 200
L = 200
D = 64
V = 100000
LANES = 16
LP = 256
_i32 = jnp.int32


def _sc_partials(seqs, ids, embT, bias_flat):
    info = plsc.get_sparse_core_info()
    nc, ns = info.num_cores, info.num_subcores
    nw = nc * ns

    mesh = plsc.VectorSubcoreMesh(core_axis_name="c", subcore_axis_name="s")

    @functools.partial(
        pl.kernel,
        mesh=mesh,
        compiler_params=pltpu.CompilerParams(
            needs_layout_passes=False, use_tc_tiling_on_sc=True),
        out_type=[
            jax.ShapeDtypeStruct((D, B, LP), jnp.float32),
            jax.ShapeDtypeStruct((B, LP), jnp.float32),
        ],
        scratch_types=[
            pltpu.VMEM((V,), jnp.float32),
            pltpu.VMEM((2, 16, L), jnp.int32),
            pltpu.VMEM((2, 16, L), jnp.int32),
            pltpu.VMEM((2, 16, LP), jnp.float32),
        ] + [pltpu.VMEM((L,), jnp.int32) for _ in range(7)]
          + [pltpu.VMEM((LP,), jnp.float32) for _ in range(7)]
          + [
            pltpu.SemaphoreType.DMA,
            pltpu.SemaphoreType.DMA,
            pltpu.SemaphoreType.DMA,
            pltpu.SemaphoreType.DMA,
        ],
    )
    def k(seq_hbm, ids_hbm, embT_hbm, bias_hbm, parts_hbm, biasg_hbm,
          col_v, sidx_v, tidx_v, part_v, *rest):
        bidx = list(rest[0:7])
        bval = list(rest[7:14])
        sem, isem, psem, csem = rest[14], rest[15], rest[16], rest[17]
        c = lax.axis_index("c")
        s = lax.axis_index("s")
        wid = s * nc + c

        bias_bs = [wid + nw * i for i in range(7)]

        d_first = c * _i32(32) + s
        pltpu.async_copy(embT_hbm.at[d_first], col_v, csem)

        def bias_stage(fire):
            for i in range(7):
                @pl.when(bias_bs[i] < B)
                def _(i=i):
                    fire(i)

        def _fire_idx(i):
            pltpu.async_copy(ids_hbm.at[bias_bs[i]],
                             bidx[i], sem)

        def _wait_idx(i):
            pltpu.make_async_copy(ids_hbm.at[bias_bs[i]],
                                  bidx[i], sem).wait()

        def _fire_val(i):
            pltpu.async_copy(bias_hbm.at[bidx[i].at[pl.ds(0, 128)]],
                             bval[i].at[pl.ds(0, 128)], sem)
            pltpu.async_copy(bias_hbm.at[bidx[i].at[pl.ds(128, 72)]],
                             bval[i].at[pl.ds(128, 72)], sem)

        def _wait_val(i):
            pltpu.make_async_copy(bias_hbm.at[bidx[i].at[pl.ds(0, 128)]],
                                  bval[i].at[pl.ds(0, 128)], sem).wait()
            pltpu.make_async_copy(bias_hbm.at[bidx[i].at[pl.ds(128, 72)]],
                                  bval[i].at[pl.ds(128, 72)], sem).wait()

        def _fire_out(i):
            pltpu.async_copy(bval[i], biasg_hbm.at[bias_bs[i]], sem)

        def _wait_out(i):
            pltpu.make_async_copy(bval[i],
                                  biasg_hbm.at[bias_bs[i]], sem).wait()

        bias_stage(_fire_idx)
        bias_stage(_wait_idx)
        bias_stage(_fire_val)
        bias_stage(_wait_val)
        bias_stage(_fire_out)
        bias_stage(_wait_out)

        NI = 8

        def chunk_b0(g):
            return pl.multiple_of(jnp.minimum(g * _i32(16), _i32(B - 16)), 8)

        def idx_copies(g, slot):
            b0 = chunk_b0(g)
            return [
                (seq_hbm.at[pl.ds(b0, 16)], sidx_v.at[slot]),
                (ids_hbm.at[pl.ds(b0, 16)], tidx_v.at[slot]),
            ]

        def quad_body_for(slot):
            def quad_body(rq, carry2):
                rows = [rq * _i32(NI) + _i32(j) for j in range(NI)]
                cc = [jnp.float32(0.0)] * NI
                cc_tail = [jnp.float32(0.0)] * NI
                for grp in range(13):
                    l0 = grp * 16 if grp < 12 else L - 16
                    xs = [plsc.load_gather(
                        col_v, [sidx_v[slot, r, pl.ds(l0, 16)]])
                        for r in rows]
                    ts = [plsc.load_gather(
                        col_v, [tidx_v[slot, r, pl.ds(l0, 16)]])
                        for r in rows]
                    pres = [plsc.cumsum(x) for x in xs]
                    cums = []
                    for j in range(NI):
                        if grp < 12:
                            cums.append(pres[j] + cc[j])
                            if grp == 11:
                                cc_tail[j] = cc[j] + pres[j][7]
                            cc[j] = cc[j] + pres[j][15]
                        else:
                            cums.append(pres[j] + cc_tail[j])
                    for j in range(NI):
                        part_v[slot, rows[j], pl.ds(l0, 16)] = cums[j] * ts[j]
                return carry2

            return quad_body

        for dpass in range(2):
            d = c * _i32(32) + _i32(16 * dpass) + s
            if dpass == 0:
                pltpu.make_async_copy(embT_hbm.at[d_first], col_v, csem).wait()
            else:
                pltpu.sync_copy(embT_hbm.at[d], col_v)
            for src_, dst_ in idx_copies(_i32(0), _i32(0)):
                pltpu.async_copy(src_, dst_, isem)

            def chunk_body(g, carry):
                slot = lax.rem(g, _i32(2))
                for src_, dst_ in idx_copies(g, slot):
                    pltpu.make_async_copy(src_, dst_, isem).wait()

                @pl.when(g < _i32(12))
                def _():
                    for src_, dst_ in idx_copies(g + 1, _i32(1) - slot):
                        pltpu.async_copy(src_, dst_, isem)

                @pl.when(g >= _i32(2))
                def _():
                    pltpu.make_async_copy(
                        part_v.at[slot],
                        parts_hbm.at[d, pl.ds(chunk_b0(g), 16)], psem).wait()

                lax.fori_loop(_i32(0), _i32(2), quad_body_for(slot), 0)
                pltpu.async_copy(
                    part_v.at[slot],
                    parts_hbm.at[d, pl.ds(chunk_b0(g), 16)], psem)
                return carry

            lax.fori_loop(_i32(0), _i32(13), chunk_body, 0)

            for gg in (11, 12):
                slot = _i32(gg % 2)
                pltpu.make_async_copy(
                    part_v.at[slot],
                    parts_hbm.at[d, pl.ds(chunk_b0(_i32(gg)), 16)],
                    psem).wait()

    return k(seqs, ids, embT, bias_flat)


def _tc_broadcast(parts, biasg):
    gi = 8

    def body(parts_ref, bias_ref, out_ref, dot_v):
        @pl.when(pl.program_id(0) == 0)
        def _():
            acc = parts_ref[0]
            for d in range(1, D):
                acc = acc + parts_ref[d]
            dot_v[...] = acc

        dot = dot_v[:, :L]
        out_ref[...] = dot[None, :, :] + bias_ref[:, :L][:, :, None]

    return pl.pallas_call(
        body,
        grid=(B // gi,),
        in_specs=[
            pl.BlockSpec((D, B, LP), lambda i: (_i32(0), _i32(0), _i32(0))),
            pl.BlockSpec((gi, LP), lambda i: (i, _i32(0))),
        ],
        out_specs=pl.BlockSpec(
            (gi, B, L), lambda i: (i, _i32(0), _i32(0))),
        out_shape=jax.ShapeDtypeStruct((B, B, L), jnp.float32),
        scratch_shapes=[pltpu.VMEM((B, LP), jnp.float32)],
    )(parts, biasg)


def kernel(item_sequences, item_ids, emb_weight, bias_weight):
    seqs = item_sequences.astype(jnp.int32)
    ids = item_ids.astype(jnp.int32)
    bias_flat = bias_weight.reshape((-1,))
    parts, biasg = _sc_partials(seqs, ids, emb_weight.T, bias_flat)
    return _tc_broadcast(parts, biasg)
